# baseline (speedup 1.0000x reference)
import jax
import jax.numpy as jnp
from jax import lax
from jax.experimental import pallas as pl
from jax.experimental.pallas import tpu as pltpu
from jax.experimental.pallas import tpu_sc as plsc

N = 100000
M = 1000
D = 128
CH = 256
NC, NS = 2, 16
NW = NC * NS
FULL = N // CH
REM = N - FULL * CH
TPW = (FULL + NW - 1) // NW
NB = 3
LA = NB - 1


def _sc_body(x_hbm, vals_hbm, idx_hbm, out_hbm,
             buf0, buf1, buf2, ib0, ib1, ib2, tab,
             sx0, sx1, sx2, si0, si1, si2, so0, so1, so2):
    bufs = (buf0, buf1, buf2)
    ibs = (ib0, ib1, ib2)
    sxs = (sx0, sx1, sx2)
    sis = (si0, si1, si2)
    sos = (so0, so1, so2)
    wid = lax.axis_index("s") * NC + lax.axis_index("c")
    pltpu.sync_copy(vals_hbm, tab)
    col128 = jnp.full((16,), D, jnp.int32)
    riota = lax.broadcasted_iota(jnp.int32, (16,), 0)

    def in_copies(t):
        cid = wid + NW * t
        s = t % NB
        return (
            pltpu.make_async_copy(idx_hbm.at[pl.ds(cid * CH, CH)], ibs[s], sis[s]),
            pltpu.make_async_copy(x_hbm.at[pl.ds(cid * CH, CH), :],
                                  bufs[s].at[:, 0:D], sxs[s]),
        )

    def out_copy(t):
        cid = wid + NW * t
        s = t % NB
        return pltpu.make_async_copy(bufs[s], out_hbm.at[pl.ds(cid * CH, CH), :], sos[s])

    def fill(t):
        s = t % NB
        for j in range(CH // 16):
            iv = ibs[s][pl.ds(j * 16, 16)]
            vals = plsc.load_gather(tab, [iv])
            plsc.store_scatter(bufs[s], [riota + j * 16, col128], vals)

    waited = set()

    def wait_out(t):
        if t < 0 or t in waited:
            return
        waited.add(t)

        @pl.when(wid + NW * t < FULL)
        def _():
            out_copy(t).wait()

    def start_in(u):
        if u >= TPW:
            return
        wait_out(u - NB)

        @pl.when(wid + NW * u < FULL)
        def _():
            c1, c2 = in_copies(u)
            c1.start()
            c2.start()

    for t in range(LA):
        start_in(t)
    for t in range(TPW):
        start_in(t + LA)

        @pl.when(wid + NW * t < FULL)
        def _proc():
            c1, c2 = in_copies(t)
            c1.wait()
            c2.wait()
            fill(t)
            out_copy(t).start()

    for t in range(TPW):
        wait_out(t)

    @pl.when(wid == FULL % NW)
    def _rem():
        pltpu.sync_copy(idx_hbm.at[pl.ds(FULL * CH, REM)], ib0.at[pl.ds(0, REM)])
        pltpu.sync_copy(x_hbm.at[pl.ds(FULL * CH, REM), :],
                        buf0.at[pl.ds(0, REM), 0:D])
        for j in range(REM // 16):
            iv = ib0[pl.ds(j * 16, 16)]
            vals = plsc.load_gather(tab, [iv])
            plsc.store_scatter(buf0, [riota + j * 16, col128], vals)
        pltpu.sync_copy(buf0.at[pl.ds(0, REM), :],
                        out_hbm.at[pl.ds(FULL * CH, REM), :])


def kernel(per_atom_property_tensor, per_molecule_values, atomic_subsystem_indices):
    vals_p = jnp.zeros((1024,), jnp.float32).at[:M].set(per_molecule_values)
    mesh = plsc.VectorSubcoreMesh(
        core_axis_name="c", subcore_axis_name="s", num_cores=NC, num_subcores=NS)
    f = pl.kernel(
        _sc_body,
        out_type=jax.ShapeDtypeStruct((N, D + 1), jnp.float32),
        mesh=mesh,
        scratch_types=(
            [pltpu.VMEM((CH, D + 1), jnp.float32)] * NB
            + [pltpu.VMEM((CH,), jnp.int32)] * NB
            + [pltpu.VMEM((1024,), jnp.float32)]
            + [pltpu.SemaphoreType.DMA] * (3 * NB)
        ),
        compiler_params=pltpu.CompilerParams(
            needs_layout_passes=False, use_tc_tiling_on_sc=False),
    )
    return f(per_atom_property_tensor, vals_p, atomic_subsystem_indices)


# speedup vs baseline: 3.8675x; 3.8675x over previous
import jax
import jax.numpy as jnp
from jax import lax
from jax.experimental import pallas as pl
from jax.experimental.pallas import tpu as pltpu
from jax.experimental.pallas import tpu_sc as plsc

N = 100000
M = 1000
D = 128
CH = 160
NC, NS = 2, 16
NW = NC * NS
FULL = N // CH
REM = N - FULL * CH
TPW = (FULL + NW - 1) // NW
NB = 3
LA = NB - 1


def _sc_body(x_hbm, vals_hbm, idx_hbm, out_hbm,
             buf0, buf1, buf2, ib0, ib1, ib2, tab,
             sx0, sx1, sx2, si0, si1, si2, so0, so1, so2):
    bufs = (buf0, buf1, buf2)
    ibs = (ib0, ib1, ib2)
    sxs = (sx0, sx1, sx2)
    sis = (si0, si1, si2)
    sos = (so0, so1, so2)
    wid = lax.axis_index("s") * NC + lax.axis_index("c")
    pltpu.sync_copy(vals_hbm, tab)
    col128 = jnp.full((16,), D, jnp.int32)
    riota = lax.broadcasted_iota(jnp.int32, (16,), 0)

    def in_copies(t):
        cid = wid + NW * t
        s = t % NB
        return (
            pltpu.make_async_copy(idx_hbm.at[pl.ds(cid * CH, CH)], ibs[s], sis[s]),
            pltpu.make_async_copy(x_hbm.at[pl.ds(cid * CH, CH), :],
                                  bufs[s].at[:, 0:D], sxs[s]),
        )

    def out_copy(t):
        cid = wid + NW * t
        s = t % NB
        return pltpu.make_async_copy(bufs[s], out_hbm.at[pl.ds(cid * CH, CH), :], sos[s])

    def fill(t):
        s = t % NB
        for j in range(CH // 16):
            iv = ibs[s][pl.ds(j * 16, 16)]
            vals = plsc.load_gather(tab, [iv])
            plsc.store_scatter(bufs[s], [riota + j * 16, col128], vals)

    waited = set()

    def wait_out(t):
        if t < 0 or t in waited:
            return
        waited.add(t)

        @pl.when(wid + NW * t < FULL)
        def _():
            out_copy(t).wait()

    def start_in(u):
        if u >= TPW:
            return
        wait_out(u - NB)

        @pl.when(wid + NW * u < FULL)
        def _():
            c1, c2 = in_copies(u)
            c1.start()
            c2.start()

    for t in range(LA):
        start_in(t)
    for t in range(TPW):
        start_in(t + LA)

        @pl.when(wid + NW * t < FULL)
        def _proc():
            c1, c2 = in_copies(t)
            c1.wait()
            c2.wait()
            fill(t)
            out_copy(t).start()

    for t in range(TPW):
        wait_out(t)


def kernel(per_atom_property_tensor, per_molecule_values, atomic_subsystem_indices):
    vals_p = jnp.zeros((1024,), jnp.float32).at[:M].set(per_molecule_values)
    mesh = plsc.VectorSubcoreMesh(
        core_axis_name="c", subcore_axis_name="s", num_cores=NC, num_subcores=NS)
    f = pl.kernel(
        _sc_body,
        out_type=jax.ShapeDtypeStruct((N, D + 1), jnp.float32),
        mesh=mesh,
        scratch_types=(
            [pltpu.VMEM((CH, D + 1), jnp.float32)] * NB
            + [pltpu.VMEM((CH,), jnp.int32)] * NB
            + [pltpu.VMEM((1024,), jnp.float32)]
            + [pltpu.SemaphoreType.DMA] * (3 * NB)
        ),
        compiler_params=pltpu.CompilerParams(
            needs_layout_passes=False),
    )
    return f(per_atom_property_tensor, vals_p, atomic_subsystem_indices)
